# speedup vs baseline: 9.4967x; 9.4967x over previous
import jax
import jax.numpy as jnp
from jax.experimental import pallas as pl
from jax.experimental.pallas import tpu as pltpu

_BW = 512


def _body(x_ref, t_ref, o_ref, carry_ref):
    g = pl.program_id(0)

    @pl.when(g == 0)
    def _init():
        carry_ref[...] = jnp.zeros_like(carry_ref)

    b = x_ref[...]
    rev = jax.lax.dot(b, t_ref[...], preferred_element_type=jnp.float32)
    o_ref[...] = rev + carry_ref[...]
    carry_ref[...] = carry_ref[...] + rev[:, 0:1]


@jax.jit
def kernel(x):
    m, n = x.shape
    nb = n // _BW
    k = jax.lax.broadcasted_iota(jnp.int32, (_BW, _BW), 0)
    j = jax.lax.broadcasted_iota(jnp.int32, (_BW, _BW), 1)
    tri = (k >= j).astype(jnp.float32)
    return pl.pallas_call(
        _body,
        grid=(nb,),
        in_specs=[
            pl.BlockSpec((m, _BW), lambda g, nb=nb: (0, nb - 1 - g)),
            pl.BlockSpec((_BW, _BW), lambda g: (0, 0)),
        ],
        out_specs=pl.BlockSpec((m, _BW), lambda g, nb=nb: (0, nb - 1 - g)),
        out_shape=jax.ShapeDtypeStruct((m, n), x.dtype),
        scratch_shapes=[pltpu.VMEM((m, 1), jnp.float32)],
    )(x, tri)
